# baseline (speedup 1.0000x reference)
import functools

import jax
import jax.numpy as jnp
from jax import lax
from jax.experimental import pallas as pl
from jax.experimental.pallas import tpu as pltpu, tpu_sc as plsc

N_NODES = 10000
N_EDGES = 320000
D = 128
N_CLASSES = 64
N_LAYERS = 5
EPS = 1e-5

NW = 32
B = 128
NBLK = 84
NBUF = 2
G = 4
NG = NBLK // G
ROUNDS = NBLK // NBUF
E_PAD = NW * NBLK * B
N_PAD = 10112
STRIPE = N_PAD // 16
ROW_F32 = jnp.float32

_mesh = plsc.VectorSubcoreMesh(
    core_axis_name="c", subcore_axis_name="s", num_cores=2, num_subcores=16)


def _deg_body(dstp_hbm, ones_hbm, zeros_hbm, out_hbm, dst_v, ones_v, sem, acc_sh):
    c = lax.axis_index("c")
    s = lax.axis_index("s")
    wid = c * 16 + s
    pltpu.sync_copy(dstp_hbm.at[wid], dst_v)
    pltpu.sync_copy(ones_hbm, ones_v)
    pltpu.sync_copy(zeros_hbm, acc_sh.at[pl.ds(s * STRIPE, STRIPE)])
    plsc.subcore_barrier()

    DEG_ASYNC = False
    if DEG_ASYNC:
        K = 8

        def blk(j, carry):
            pltpu.async_copy(ones_v, acc_sh.at[dst_v.at[j]], sem, add=True)

            @pl.when(j >= K)
            def _():
                pltpu.make_async_copy(ones_v, acc_sh.at[dst_v.at[j]],
                                      sem).wait()
            return carry

        lax.fori_loop(0, NBLK, blk, 0)
        for _ in range(K):
            pltpu.make_async_copy(ones_v, acc_sh.at[dst_v.at[0]], sem).wait()
    else:
        def blk(j, carry):
            pltpu.sync_copy(ones_v, acc_sh.at[dst_v.at[j]], add=True)
            return carry

        lax.fori_loop(0, NBLK, blk, 0)
    plsc.subcore_barrier()
    pltpu.sync_copy(
        acc_sh.at[pl.ds(s * STRIPE, STRIPE)],
        out_hbm.at[pl.ds((c * N_PAD + s * STRIPE), STRIPE)],
    )


_deg_call = pl.kernel(
    _deg_body,
    out_type=jax.ShapeDtypeStruct((2 * N_PAD, 16), ROW_F32),
    mesh=_mesh,
    scratch_types=[
        pltpu.VMEM((NBLK, B), jnp.int32),
        pltpu.VMEM((B, 16), ROW_F32),
        pltpu.SemaphoreType.DMA,
        pltpu.VMEM_SHARED((N_PAD, 16), ROW_F32),
    ],
)


def _agg_body(t_hbm, srcp_hbm, dstp_hbm, zeros_hbm, out_hbm,
              sidx_v, didx_v, rows0, rows1,
              g0, g1, s0, s1, isem, acc_sh):
    c = lax.axis_index("c")
    s = lax.axis_index("s")
    wid = c * 16 + s
    rows = (rows0, rows1)
    gsem = (g0, g1)
    ssem = (s0, s1)

    pltpu.sync_copy(srcp_hbm.at[wid, pl.ds(0, G)], sidx_v.at[pl.ds(0, G)])
    pltpu.sync_copy(dstp_hbm.at[wid, pl.ds(0, G)], didx_v.at[pl.ds(0, G)])
    pltpu.async_copy(srcp_hbm.at[wid, pl.ds(G, G)], sidx_v.at[pl.ds(G, G)],
                     isem)
    pltpu.async_copy(dstp_hbm.at[wid, pl.ds(G, G)], didx_v.at[pl.ds(G, G)],
                     isem)

    pltpu.sync_copy(zeros_hbm, acc_sh.at[pl.ds(s * STRIPE, STRIPE)])
    plsc.subcore_barrier()

    RPG = G // NBUF

    RING = False
    if RING:
        for b in range(NBUF):
            pltpu.async_copy(t_hbm.at[sidx_v.at[b]], rows[b], gsem[b])

    def rnd(i, carry):
        g = i // RPG
        phase = i % RPG

        @pl.when(jnp.logical_and(phase == RPG - 1, g + 1 < NG))
        def _():
            pltpu.make_async_copy(srcp_hbm.at[wid, pl.ds(0, G)],
                                  sidx_v.at[pl.ds(0, G)], isem).wait()
            pltpu.make_async_copy(dstp_hbm.at[wid, pl.ds(0, G)],
                                  didx_v.at[pl.ds(0, G)], isem).wait()

        @pl.when(jnp.logical_and(phase == 0,
                                 jnp.logical_and(g >= 1, g + 1 < NG)))
        def _():
            off = pl.multiple_of(((g + 1) % 2) * G, G)
            src_off = pl.multiple_of((g + 1) * G, G)
            pltpu.async_copy(srcp_hbm.at[wid, pl.ds(src_off, G)],
                             sidx_v.at[pl.ds(off, G)], isem)
            pltpu.async_copy(dstp_hbm.at[wid, pl.ds(src_off, G)],
                             didx_v.at[pl.ds(off, G)], isem)

        if RING:
            for b in range(NBUF):
                j = i * NBUF + b
                jj = j % (2 * G)
                pltpu.make_async_copy(t_hbm.at[sidx_v.at[jj]], rows[b],
                                      gsem[b]).wait()
                pltpu.async_copy(rows[b], acc_sh.at[didx_v.at[jj]], ssem[b],
                                 add=True)

            @pl.when(i + 1 < ROUNDS)
            def _():
                for b in range(NBUF):
                    j = i * NBUF + b
                    jj = j % (2 * G)
                    jn = (j + NBUF) % (2 * G)
                    pltpu.make_async_copy(rows[b], acc_sh.at[didx_v.at[jj]],
                                          ssem[b]).wait()
                    pltpu.async_copy(t_hbm.at[sidx_v.at[jn]], rows[b],
                                     gsem[b])
        else:
            for b in range(NBUF):
                j = i * NBUF + b
                jj = j % (2 * G)
                pltpu.async_copy(t_hbm.at[sidx_v.at[jj]], rows[b], gsem[b])
                pltpu.make_async_copy(t_hbm.at[sidx_v.at[jj]], rows[b],
                                      gsem[b]).wait()
                pltpu.sync_copy(rows[b], acc_sh.at[didx_v.at[jj]], add=True)
        return carry

    lax.fori_loop(0, ROUNDS, rnd, 0)
    if RING:
        for b in range(NBUF):
            jj = ((ROUNDS - 1) * NBUF + b) % (2 * G)
            pltpu.make_async_copy(rows[b], acc_sh.at[didx_v.at[jj]],
                                  ssem[b]).wait()
    plsc.subcore_barrier()
    pltpu.sync_copy(
        acc_sh.at[pl.ds(s * STRIPE, STRIPE)],
        out_hbm.at[pl.ds((c * N_PAD + s * STRIPE), STRIPE)],
    )


_agg_call = pl.kernel(
    _agg_body,
    out_type=jax.ShapeDtypeStruct((2 * N_PAD, D), ROW_F32),
    mesh=_mesh,
    scratch_types=[
        pltpu.VMEM((2 * G, B), jnp.int32),
        pltpu.VMEM((2 * G, B), jnp.int32),
        pltpu.VMEM((B, D), ROW_F32),
        pltpu.VMEM((B, D), ROW_F32),
        pltpu.SemaphoreType.DMA,
        pltpu.SemaphoreType.DMA,
        pltpu.SemaphoreType.DMA,
        pltpu.SemaphoreType.DMA,
        pltpu.SemaphoreType.DMA,
        pltpu.VMEM_SHARED((N_PAD, D), ROW_F32),
    ],
)


R = 1000


def _tca_body(degp_ref, x_ref, w_ref, t_ref, dinv_ref):
    dsum = degp_ref[0] + degp_ref[1]
    deg = jnp.sum(dsum, axis=-1, keepdims=True) * (1.0 / 16.0)
    dinv = lax.rsqrt(deg)
    t = jnp.dot(x_ref[...], w_ref[...], preferred_element_type=jnp.float32)
    t_ref[...] = t * dinv
    dinv_ref[...] = dinv


def _tc_first(degp, x, w0):
    return pl.pallas_call(
        _tca_body,
        grid=(N_NODES // R,),
        in_specs=[
            pl.BlockSpec((2, R, 16), lambda i: (0, i, 0)),
            pl.BlockSpec((R, D), lambda i: (i, 0)),
            pl.BlockSpec((D, D), lambda i: (0, 0)),
        ],
        out_specs=[
            pl.BlockSpec((R, D), lambda i: (i, 0)),
            pl.BlockSpec((R, 1), lambda i: (i, 0)),
        ],
        out_shape=[
            jax.ShapeDtypeStruct((N_NODES, D), jnp.float32),
            jax.ShapeDtypeStruct((N_NODES, 1), jnp.float32),
        ],
    )(degp, x, w0)


def _ln_relu(p_ref, dinv_ref, b_ref, g_ref, be_ref):
    h = (p_ref[0] + p_ref[1]) * dinv_ref[...] + b_ref[...]
    mu = jnp.mean(h, axis=-1, keepdims=True)
    hc = h - mu
    var = jnp.mean(hc * hc, axis=-1, keepdims=True)
    h = hc * lax.rsqrt(var + EPS) * g_ref[...] + be_ref[...]
    return jnp.maximum(h, 0.0)


def _tcb_body(p_ref, dinv_ref, b_ref, g_ref, be_ref, w_ref, out_ref):
    h = _ln_relu(p_ref, dinv_ref, b_ref, g_ref, be_ref)
    t = jnp.dot(h, w_ref[...], preferred_element_type=jnp.float32)
    out_ref[...] = t * dinv_ref[...]


def _tc_mid(aggp, dinv, b, g, be, w):
    return pl.pallas_call(
        _tcb_body,
        grid=(N_NODES // R,),
        in_specs=[
            pl.BlockSpec((2, R, D), lambda i: (0, i, 0)),
            pl.BlockSpec((R, 1), lambda i: (i, 0)),
            pl.BlockSpec((1, D), lambda i: (0, 0)),
            pl.BlockSpec((1, D), lambda i: (0, 0)),
            pl.BlockSpec((1, D), lambda i: (0, 0)),
            pl.BlockSpec((D, D), lambda i: (0, 0)),
        ],
        out_specs=pl.BlockSpec((R, D), lambda i: (i, 0)),
        out_shape=jax.ShapeDtypeStruct((N_NODES, D), jnp.float32),
    )(aggp, dinv, b, g, be, w)


def _tcc_body(p_ref, dinv_ref, b_ref, g_ref, be_ref, w_ref, bo_ref, out_ref):
    h = _ln_relu(p_ref, dinv_ref, b_ref, g_ref, be_ref)
    out_ref[...] = (
        jnp.dot(h, w_ref[...], preferred_element_type=jnp.float32) + bo_ref[...]
    )


def _tc_last(aggp, dinv, b, g, be, wout, bout):
    return pl.pallas_call(
        _tcc_body,
        grid=(N_NODES // R,),
        in_specs=[
            pl.BlockSpec((2, R, D), lambda i: (0, i, 0)),
            pl.BlockSpec((R, 1), lambda i: (i, 0)),
            pl.BlockSpec((1, D), lambda i: (0, 0)),
            pl.BlockSpec((1, D), lambda i: (0, 0)),
            pl.BlockSpec((1, D), lambda i: (0, 0)),
            pl.BlockSpec((D, N_CLASSES), lambda i: (0, 0)),
            pl.BlockSpec((1, N_CLASSES), lambda i: (0, 0)),
        ],
        out_specs=pl.BlockSpec((R, N_CLASSES), lambda i: (i, 0)),
        out_shape=jax.ShapeDtypeStruct((N_NODES, N_CLASSES), jnp.float32),
    )(aggp, dinv, b, g, be, wout, bout)


def kernel(x, edge_index, Ws, bs, gammas, betas, Wout, bout):
    src = edge_index[0].astype(jnp.int32)
    dst = edge_index[1].astype(jnp.int32)
    loop = jnp.arange(N_NODES, dtype=jnp.int32)
    n_dummy = E_PAD - N_EDGES - N_NODES
    src_a = jnp.concatenate(
        [src, loop, jnp.zeros((n_dummy,), jnp.int32)]).reshape(NW, NBLK, B)
    dst_a = jnp.concatenate(
        [dst, loop, jnp.full((n_dummy,), N_NODES, jnp.int32)]).reshape(NW, NBLK, B)

    ones_r = jnp.ones((B, 16), ROW_F32)
    zer16 = jnp.zeros((STRIPE, 16), ROW_F32)
    zer128 = jnp.zeros((STRIPE, D), ROW_F32)

    degp = _deg_call(dst_a, ones_r, zer16).reshape(2, N_PAD, 16)
    t, dinv = _tc_first(degp, x, Ws[0])

    aggp = None
    for i in range(N_LAYERS):
        aggp = _agg_call(t, src_a, dst_a, zer128).reshape(2, N_PAD, D)
        if i + 1 < N_LAYERS:
            t = _tc_mid(aggp, dinv, bs[i][None, :], gammas[i][None, :],
                        betas[i][None, :], Ws[i + 1])

    i = N_LAYERS - 1
    return _tc_last(aggp, dinv, bs[i][None, :], gammas[i][None, :],
                    betas[i][None, :], Wout, bout[None, :])


# speedup vs baseline: 1.0484x; 1.0484x over previous
import functools

import jax
import jax.numpy as jnp
from jax import lax
from jax.experimental import pallas as pl
from jax.experimental.pallas import tpu as pltpu, tpu_sc as plsc

N_NODES = 10000
N_EDGES = 320000
D = 128
N_CLASSES = 64
N_LAYERS = 5
EPS = 1e-5

NW = 32
B = 128
NBLK = 84
NBUF = 2
G = 4
NG = NBLK // G
ROUNDS = NBLK // NBUF
E_PAD = NW * NBLK * B
N_PAD = 10112
STRIPE = N_PAD // 16
ROW_F32 = jnp.float32

_mesh = plsc.VectorSubcoreMesh(
    core_axis_name="c", subcore_axis_name="s", num_cores=2, num_subcores=16)


def _deg_body(dstp_hbm, ones_hbm, zeros_hbm, out_hbm, dst_v, ones_v, sem, acc_sh):
    c = lax.axis_index("c")
    s = lax.axis_index("s")
    wid = c * 16 + s
    pltpu.sync_copy(dstp_hbm.at[wid], dst_v)
    pltpu.sync_copy(ones_hbm, ones_v)
    pltpu.sync_copy(zeros_hbm, acc_sh.at[pl.ds(s * STRIPE, STRIPE)])
    plsc.subcore_barrier()

    DEG_ASYNC = False
    if DEG_ASYNC:
        K = 8

        def blk(j, carry):
            pltpu.async_copy(ones_v, acc_sh.at[dst_v.at[j]], sem, add=True)

            @pl.when(j >= K)
            def _():
                pltpu.make_async_copy(ones_v, acc_sh.at[dst_v.at[j]],
                                      sem).wait()
            return carry

        lax.fori_loop(0, NBLK, blk, 0)
        for _ in range(K):
            pltpu.make_async_copy(ones_v, acc_sh.at[dst_v.at[0]], sem).wait()
    else:
        def blk(j, carry):
            pltpu.sync_copy(ones_v, acc_sh.at[dst_v.at[j]], add=True)
            return carry

        lax.fori_loop(0, NBLK, blk, 0)
    plsc.subcore_barrier()
    pltpu.sync_copy(
        acc_sh.at[pl.ds(s * STRIPE, STRIPE)],
        out_hbm.at[pl.ds((c * N_PAD + s * STRIPE), STRIPE)],
    )


_deg_call = pl.kernel(
    _deg_body,
    out_type=jax.ShapeDtypeStruct((2 * N_PAD, 16), ROW_F32),
    mesh=_mesh,
    scratch_types=[
        pltpu.VMEM((NBLK, B), jnp.int32),
        pltpu.VMEM((B, 16), ROW_F32),
        pltpu.SemaphoreType.DMA,
        pltpu.VMEM_SHARED((N_PAD, 16), ROW_F32),
    ],
)


def _agg_body(t_hbm, srcp_hbm, dstp_hbm, zeros_hbm, out_hbm,
              sidx_v, didx_v, rows0, rows1,
              g0, g1, s0, s1, isem, acc_sh):
    c = lax.axis_index("c")
    s = lax.axis_index("s")
    wid = c * 16 + s
    rows = (rows0, rows1)
    gsem = (g0, g1)
    ssem = (s0, s1)

    pltpu.sync_copy(srcp_hbm.at[wid, pl.ds(0, G)], sidx_v.at[pl.ds(0, G)])
    pltpu.sync_copy(dstp_hbm.at[wid, pl.ds(0, G)], didx_v.at[pl.ds(0, G)])
    pltpu.async_copy(srcp_hbm.at[wid, pl.ds(G, G)], sidx_v.at[pl.ds(G, G)],
                     isem)
    pltpu.async_copy(dstp_hbm.at[wid, pl.ds(G, G)], didx_v.at[pl.ds(G, G)],
                     isem)

    pltpu.sync_copy(zeros_hbm, acc_sh.at[pl.ds(s * STRIPE, STRIPE)])
    plsc.subcore_barrier()

    RPG = G // NBUF

    RING = True
    if RING:
        for b in range(NBUF):
            pltpu.async_copy(t_hbm.at[sidx_v.at[b]], rows[b], gsem[b])

    def rnd(i, carry):
        g = i // RPG
        phase = i % RPG

        @pl.when(jnp.logical_and(phase == RPG - 1, g + 1 < NG))
        def _():
            pltpu.make_async_copy(srcp_hbm.at[wid, pl.ds(0, G)],
                                  sidx_v.at[pl.ds(0, G)], isem).wait()
            pltpu.make_async_copy(dstp_hbm.at[wid, pl.ds(0, G)],
                                  didx_v.at[pl.ds(0, G)], isem).wait()

        @pl.when(jnp.logical_and(phase == 0,
                                 jnp.logical_and(g >= 1, g + 1 < NG)))
        def _():
            off = pl.multiple_of(((g + 1) % 2) * G, G)
            src_off = pl.multiple_of((g + 1) * G, G)
            pltpu.async_copy(srcp_hbm.at[wid, pl.ds(src_off, G)],
                             sidx_v.at[pl.ds(off, G)], isem)
            pltpu.async_copy(dstp_hbm.at[wid, pl.ds(src_off, G)],
                             didx_v.at[pl.ds(off, G)], isem)

        if RING:
            for b in range(NBUF):
                j = i * NBUF + b
                jj = j % (2 * G)
                pltpu.make_async_copy(t_hbm.at[sidx_v.at[jj]], rows[b],
                                      gsem[b]).wait()
                pltpu.async_copy(rows[b], acc_sh.at[didx_v.at[jj]], ssem[b],
                                 add=True)

            @pl.when(i + 1 < ROUNDS)
            def _():
                for b in range(NBUF):
                    j = i * NBUF + b
                    jj = j % (2 * G)
                    jn = (j + NBUF) % (2 * G)
                    pltpu.make_async_copy(rows[b], acc_sh.at[didx_v.at[jj]],
                                          ssem[b]).wait()
                    pltpu.async_copy(t_hbm.at[sidx_v.at[jn]], rows[b],
                                     gsem[b])
        else:
            for b in range(NBUF):
                j = i * NBUF + b
                jj = j % (2 * G)
                pltpu.async_copy(t_hbm.at[sidx_v.at[jj]], rows[b], gsem[b])
                pltpu.make_async_copy(t_hbm.at[sidx_v.at[jj]], rows[b],
                                      gsem[b]).wait()
                pltpu.sync_copy(rows[b], acc_sh.at[didx_v.at[jj]], add=True)
        return carry

    lax.fori_loop(0, ROUNDS, rnd, 0)
    if RING:
        for b in range(NBUF):
            jj = ((ROUNDS - 1) * NBUF + b) % (2 * G)
            pltpu.make_async_copy(rows[b], acc_sh.at[didx_v.at[jj]],
                                  ssem[b]).wait()
    plsc.subcore_barrier()
    pltpu.sync_copy(
        acc_sh.at[pl.ds(s * STRIPE, STRIPE)],
        out_hbm.at[pl.ds((c * N_PAD + s * STRIPE), STRIPE)],
    )


_agg_call = pl.kernel(
    _agg_body,
    out_type=jax.ShapeDtypeStruct((2 * N_PAD, D), ROW_F32),
    mesh=_mesh,
    scratch_types=[
        pltpu.VMEM((2 * G, B), jnp.int32),
        pltpu.VMEM((2 * G, B), jnp.int32),
        pltpu.VMEM((B, D), ROW_F32),
        pltpu.VMEM((B, D), ROW_F32),
        pltpu.SemaphoreType.DMA,
        pltpu.SemaphoreType.DMA,
        pltpu.SemaphoreType.DMA,
        pltpu.SemaphoreType.DMA,
        pltpu.SemaphoreType.DMA,
        pltpu.VMEM_SHARED((N_PAD, D), ROW_F32),
    ],
)


R = 1000


def _tca_body(degp_ref, x_ref, w_ref, t_ref, dinv_ref):
    dsum = degp_ref[0] + degp_ref[1]
    deg = jnp.sum(dsum, axis=-1, keepdims=True) * (1.0 / 16.0)
    dinv = lax.rsqrt(deg)
    t = jnp.dot(x_ref[...], w_ref[...], preferred_element_type=jnp.float32)
    t_ref[...] = t * dinv
    dinv_ref[...] = dinv


def _tc_first(degp, x, w0):
    return pl.pallas_call(
        _tca_body,
        grid=(N_NODES // R,),
        in_specs=[
            pl.BlockSpec((2, R, 16), lambda i: (0, i, 0)),
            pl.BlockSpec((R, D), lambda i: (i, 0)),
            pl.BlockSpec((D, D), lambda i: (0, 0)),
        ],
        out_specs=[
            pl.BlockSpec((R, D), lambda i: (i, 0)),
            pl.BlockSpec((R, 1), lambda i: (i, 0)),
        ],
        out_shape=[
            jax.ShapeDtypeStruct((N_NODES, D), jnp.float32),
            jax.ShapeDtypeStruct((N_NODES, 1), jnp.float32),
        ],
    )(degp, x, w0)


def _ln_relu(p_ref, dinv_ref, b_ref, g_ref, be_ref):
    h = (p_ref[0] + p_ref[1]) * dinv_ref[...] + b_ref[...]
    mu = jnp.mean(h, axis=-1, keepdims=True)
    hc = h - mu
    var = jnp.mean(hc * hc, axis=-1, keepdims=True)
    h = hc * lax.rsqrt(var + EPS) * g_ref[...] + be_ref[...]
    return jnp.maximum(h, 0.0)


def _tcb_body(p_ref, dinv_ref, b_ref, g_ref, be_ref, w_ref, out_ref):
    h = _ln_relu(p_ref, dinv_ref, b_ref, g_ref, be_ref)
    t = jnp.dot(h, w_ref[...], preferred_element_type=jnp.float32)
    out_ref[...] = t * dinv_ref[...]


def _tc_mid(aggp, dinv, b, g, be, w):
    return pl.pallas_call(
        _tcb_body,
        grid=(N_NODES // R,),
        in_specs=[
            pl.BlockSpec((2, R, D), lambda i: (0, i, 0)),
            pl.BlockSpec((R, 1), lambda i: (i, 0)),
            pl.BlockSpec((1, D), lambda i: (0, 0)),
            pl.BlockSpec((1, D), lambda i: (0, 0)),
            pl.BlockSpec((1, D), lambda i: (0, 0)),
            pl.BlockSpec((D, D), lambda i: (0, 0)),
        ],
        out_specs=pl.BlockSpec((R, D), lambda i: (i, 0)),
        out_shape=jax.ShapeDtypeStruct((N_NODES, D), jnp.float32),
    )(aggp, dinv, b, g, be, w)


def _tcc_body(p_ref, dinv_ref, b_ref, g_ref, be_ref, w_ref, bo_ref, out_ref):
    h = _ln_relu(p_ref, dinv_ref, b_ref, g_ref, be_ref)
    out_ref[...] = (
        jnp.dot(h, w_ref[...], preferred_element_type=jnp.float32) + bo_ref[...]
    )


def _tc_last(aggp, dinv, b, g, be, wout, bout):
    return pl.pallas_call(
        _tcc_body,
        grid=(N_NODES // R,),
        in_specs=[
            pl.BlockSpec((2, R, D), lambda i: (0, i, 0)),
            pl.BlockSpec((R, 1), lambda i: (i, 0)),
            pl.BlockSpec((1, D), lambda i: (0, 0)),
            pl.BlockSpec((1, D), lambda i: (0, 0)),
            pl.BlockSpec((1, D), lambda i: (0, 0)),
            pl.BlockSpec((D, N_CLASSES), lambda i: (0, 0)),
            pl.BlockSpec((1, N_CLASSES), lambda i: (0, 0)),
        ],
        out_specs=pl.BlockSpec((R, N_CLASSES), lambda i: (i, 0)),
        out_shape=jax.ShapeDtypeStruct((N_NODES, N_CLASSES), jnp.float32),
    )(aggp, dinv, b, g, be, wout, bout)


def kernel(x, edge_index, Ws, bs, gammas, betas, Wout, bout):
    src = edge_index[0].astype(jnp.int32)
    dst = edge_index[1].astype(jnp.int32)
    loop = jnp.arange(N_NODES, dtype=jnp.int32)
    n_dummy = E_PAD - N_EDGES - N_NODES
    dummy_dst = N_NODES + (jnp.arange(n_dummy, dtype=jnp.int32)
                           % (N_PAD - N_NODES))
    src_a = jnp.concatenate(
        [src, loop, jnp.zeros((n_dummy,), jnp.int32)]).reshape(NW, NBLK, B)
    dst_a = jnp.concatenate(
        [dst, loop, dummy_dst]).reshape(NW, NBLK, B)

    ones_r = jnp.ones((B, 16), ROW_F32)
    zer16 = jnp.zeros((STRIPE, 16), ROW_F32)
    zer128 = jnp.zeros((STRIPE, D), ROW_F32)

    degp = _deg_call(dst_a, ones_r, zer16).reshape(2, N_PAD, 16)
    t, dinv = _tc_first(degp, x, Ws[0])

    aggp = None
    for i in range(N_LAYERS):
        aggp = _agg_call(t, src_a, dst_a, zer128).reshape(2, N_PAD, D)
        if i + 1 < N_LAYERS:
            t = _tc_mid(aggp, dinv, bs[i][None, :], gammas[i][None, :],
                        betas[i][None, :], Ws[i + 1])

    i = N_LAYERS - 1
    return _tc_last(aggp, dinv, bs[i][None, :], gammas[i][None, :],
                    betas[i][None, :], Wout, bout[None, :])


# speedup vs baseline: 4.8125x; 4.5905x over previous
import functools

import jax
import jax.numpy as jnp
from jax import lax
from jax.experimental import pallas as pl
from jax.experimental.pallas import tpu as pltpu, tpu_sc as plsc

N_NODES = 10000
N_EDGES = 320000
D = 128
N_CLASSES = 64
N_LAYERS = 5
EPS = 1e-5

NW = 32
B = 128
NBLK = 84
NBUF = 2
G = 4
NG = NBLK // G
ROUNDS = NBLK // NBUF
E_PAD = NW * NBLK * B
N_PAD = 10112
STRIPE = N_PAD // 16
ROW_F32 = jnp.float32

_mesh = plsc.VectorSubcoreMesh(
    core_axis_name="c", subcore_axis_name="s", num_cores=2, num_subcores=16)


def _deg_body(dstp_hbm, ones_hbm, zeros_hbm, out_hbm, dst_v, ones_v, sem, acc_sh):
    c = lax.axis_index("c")
    s = lax.axis_index("s")
    wid = c * 16 + s
    pltpu.sync_copy(dstp_hbm.at[wid], dst_v)
    pltpu.sync_copy(ones_hbm, ones_v)
    pltpu.sync_copy(zeros_hbm, acc_sh.at[pl.ds(s * STRIPE, STRIPE)])
    plsc.subcore_barrier()

    DEG_ASYNC = False
    if DEG_ASYNC:
        K = 8

        def blk(j, carry):
            pltpu.async_copy(ones_v, acc_sh.at[dst_v.at[j]], sem, add=True)

            @pl.when(j >= K)
            def _():
                pltpu.make_async_copy(ones_v, acc_sh.at[dst_v.at[j]],
                                      sem).wait()
            return carry

        lax.fori_loop(0, NBLK, blk, 0)
        for _ in range(K):
            pltpu.make_async_copy(ones_v, acc_sh.at[dst_v.at[0]], sem).wait()
    else:
        def blk(j, carry):
            pltpu.sync_copy(ones_v, acc_sh.at[dst_v.at[j]], add=True)
            return carry

        lax.fori_loop(0, NBLK, blk, 0)
    plsc.subcore_barrier()
    pltpu.sync_copy(
        acc_sh.at[pl.ds(s * STRIPE, STRIPE)],
        out_hbm.at[pl.ds((c * N_PAD + s * STRIPE), STRIPE)],
    )


_deg_call = pl.kernel(
    _deg_body,
    out_type=jax.ShapeDtypeStruct((2 * N_PAD, 16), ROW_F32),
    mesh=_mesh,
    scratch_types=[
        pltpu.VMEM((NBLK, B), jnp.int32),
        pltpu.VMEM((B, 16), ROW_F32),
        pltpu.SemaphoreType.DMA,
        pltpu.VMEM_SHARED((N_PAD, 16), ROW_F32),
    ],
)


def _agg_body(t_hbm, srcp_hbm, dstp_hbm, zeros_hbm, out_hbm,
              sidx_v, didx_v, rows0, rows1,
              g0, g1, s0, s1, isem, acc_sh):
    c = lax.axis_index("c")
    s = lax.axis_index("s")
    wid = c * 16 + s
    rows = (rows0, rows1)
    gsem = (g0, g1)
    ssem = (s0, s1)

    pltpu.sync_copy(srcp_hbm.at[wid, pl.ds(0, G)], sidx_v.at[pl.ds(0, G)])
    pltpu.sync_copy(dstp_hbm.at[wid, pl.ds(0, G)], didx_v.at[pl.ds(0, G)])
    pltpu.async_copy(srcp_hbm.at[wid, pl.ds(G, G)], sidx_v.at[pl.ds(G, G)],
                     isem)
    pltpu.async_copy(dstp_hbm.at[wid, pl.ds(G, G)], didx_v.at[pl.ds(G, G)],
                     isem)

    pltpu.sync_copy(zeros_hbm, acc_sh.at[pl.ds(s * STRIPE, STRIPE)])
    plsc.subcore_barrier()

    RPG = G // NBUF

    RING = True
    if RING:
        for b in range(NBUF):
            pltpu.async_copy(t_hbm.at[sidx_v.at[b]], rows[b], gsem[b])

    def rnd(i, carry):
        g = i // RPG
        phase = i % RPG

        @pl.when(jnp.logical_and(phase == RPG - 1, g + 1 < NG))
        def _():
            pltpu.make_async_copy(srcp_hbm.at[wid, pl.ds(0, G)],
                                  sidx_v.at[pl.ds(0, G)], isem).wait()
            pltpu.make_async_copy(dstp_hbm.at[wid, pl.ds(0, G)],
                                  didx_v.at[pl.ds(0, G)], isem).wait()

        @pl.when(jnp.logical_and(phase == 0,
                                 jnp.logical_and(g >= 1, g + 1 < NG)))
        def _():
            off = pl.multiple_of(((g + 1) % 2) * G, G)
            src_off = pl.multiple_of((g + 1) * G, G)
            pltpu.async_copy(srcp_hbm.at[wid, pl.ds(src_off, G)],
                             sidx_v.at[pl.ds(off, G)], isem)
            pltpu.async_copy(dstp_hbm.at[wid, pl.ds(src_off, G)],
                             didx_v.at[pl.ds(off, G)], isem)

        if RING:
            for b in range(NBUF):
                j = i * NBUF + b
                jj = j % (2 * G)
                pltpu.make_async_copy(t_hbm.at[sidx_v.at[jj]], rows[b],
                                      gsem[b]).wait()
                pltpu.async_copy(rows[b], acc_sh.at[didx_v.at[jj]], ssem[b],
                                 add=True)

            @pl.when(i + 1 < ROUNDS)
            def _():
                for b in range(NBUF):
                    j = i * NBUF + b
                    jj = j % (2 * G)
                    jn = (j + NBUF) % (2 * G)
                    pltpu.make_async_copy(rows[b], acc_sh.at[didx_v.at[jj]],
                                          ssem[b]).wait()
                    pltpu.async_copy(t_hbm.at[sidx_v.at[jn]], rows[b],
                                     gsem[b])
        else:
            for b in range(NBUF):
                j = i * NBUF + b
                jj = j % (2 * G)
                pltpu.async_copy(t_hbm.at[sidx_v.at[jj]], rows[b], gsem[b])
                pltpu.make_async_copy(t_hbm.at[sidx_v.at[jj]], rows[b],
                                      gsem[b]).wait()
                pltpu.sync_copy(rows[b], acc_sh.at[didx_v.at[jj]], add=True)
        return carry

    lax.fori_loop(0, ROUNDS, rnd, 0)
    if RING:
        for b in range(NBUF):
            jj = ((ROUNDS - 1) * NBUF + b) % (2 * G)
            pltpu.make_async_copy(rows[b], acc_sh.at[didx_v.at[jj]],
                                  ssem[b]).wait()
    plsc.subcore_barrier()
    pltpu.sync_copy(
        acc_sh.at[pl.ds(s * STRIPE, STRIPE)],
        out_hbm.at[pl.ds((c * N_PAD + s * STRIPE), STRIPE)],
    )


_agg_call = pl.kernel(
    _agg_body,
    out_type=jax.ShapeDtypeStruct((2 * N_PAD, D), ROW_F32),
    mesh=_mesh,
    scratch_types=[
        pltpu.VMEM((2 * G, B), jnp.int32),
        pltpu.VMEM((2 * G, B), jnp.int32),
        pltpu.VMEM((B, D), ROW_F32),
        pltpu.VMEM((B, D), ROW_F32),
        pltpu.SemaphoreType.DMA,
        pltpu.SemaphoreType.DMA,
        pltpu.SemaphoreType.DMA,
        pltpu.SemaphoreType.DMA,
        pltpu.SemaphoreType.DMA,
        pltpu.VMEM_SHARED((N_PAD, D), ROW_F32),
    ],
)


R = 1000


def _tca_body(degp_ref, x_ref, w_ref, t_ref, dinv_ref):
    dsum = degp_ref[0] + degp_ref[1]
    deg = jnp.sum(dsum, axis=-1, keepdims=True) * (1.0 / 16.0)
    dinv = lax.rsqrt(deg)
    t = jnp.dot(x_ref[...], w_ref[...], preferred_element_type=jnp.float32)
    t_ref[...] = t * dinv
    dinv_ref[...] = dinv


def _tc_first(degp, x, w0):
    return pl.pallas_call(
        _tca_body,
        grid=(N_NODES // R,),
        in_specs=[
            pl.BlockSpec((2, R, 16), lambda i: (0, i, 0)),
            pl.BlockSpec((R, D), lambda i: (i, 0)),
            pl.BlockSpec((D, D), lambda i: (0, 0)),
        ],
        out_specs=[
            pl.BlockSpec((R, D), lambda i: (i, 0)),
            pl.BlockSpec((R, 1), lambda i: (i, 0)),
        ],
        out_shape=[
            jax.ShapeDtypeStruct((N_NODES, D), jnp.float32),
            jax.ShapeDtypeStruct((N_NODES, 1), jnp.float32),
        ],
    )(degp, x, w0)


def _ln_relu(p_ref, dinv_ref, b_ref, g_ref, be_ref):
    h = (p_ref[0] + p_ref[1]) * dinv_ref[...] + b_ref[...]
    mu = jnp.mean(h, axis=-1, keepdims=True)
    hc = h - mu
    var = jnp.mean(hc * hc, axis=-1, keepdims=True)
    h = hc * lax.rsqrt(var + EPS) * g_ref[...] + be_ref[...]
    return jnp.maximum(h, 0.0)


def _tcb_body(p_ref, dinv_ref, b_ref, g_ref, be_ref, w_ref, out_ref):
    h = _ln_relu(p_ref, dinv_ref, b_ref, g_ref, be_ref)
    t = jnp.dot(h, w_ref[...], preferred_element_type=jnp.float32)
    out_ref[...] = t * dinv_ref[...]


def _tc_mid(aggp, dinv, b, g, be, w):
    return pl.pallas_call(
        _tcb_body,
        grid=(N_NODES // R,),
        in_specs=[
            pl.BlockSpec((2, R, D), lambda i: (0, i, 0)),
            pl.BlockSpec((R, 1), lambda i: (i, 0)),
            pl.BlockSpec((1, D), lambda i: (0, 0)),
            pl.BlockSpec((1, D), lambda i: (0, 0)),
            pl.BlockSpec((1, D), lambda i: (0, 0)),
            pl.BlockSpec((D, D), lambda i: (0, 0)),
        ],
        out_specs=pl.BlockSpec((R, D), lambda i: (i, 0)),
        out_shape=jax.ShapeDtypeStruct((N_NODES, D), jnp.float32),
    )(aggp, dinv, b, g, be, w)


def _tcc_body(p_ref, dinv_ref, b_ref, g_ref, be_ref, w_ref, bo_ref, out_ref):
    h = _ln_relu(p_ref, dinv_ref, b_ref, g_ref, be_ref)
    out_ref[...] = (
        jnp.dot(h, w_ref[...], preferred_element_type=jnp.float32) + bo_ref[...]
    )


def _tc_last(aggp, dinv, b, g, be, wout, bout):
    return pl.pallas_call(
        _tcc_body,
        grid=(N_NODES // R,),
        in_specs=[
            pl.BlockSpec((2, R, D), lambda i: (0, i, 0)),
            pl.BlockSpec((R, 1), lambda i: (i, 0)),
            pl.BlockSpec((1, D), lambda i: (0, 0)),
            pl.BlockSpec((1, D), lambda i: (0, 0)),
            pl.BlockSpec((1, D), lambda i: (0, 0)),
            pl.BlockSpec((D, N_CLASSES), lambda i: (0, 0)),
            pl.BlockSpec((1, N_CLASSES), lambda i: (0, 0)),
        ],
        out_specs=pl.BlockSpec((R, N_CLASSES), lambda i: (i, 0)),
        out_shape=jax.ShapeDtypeStruct((N_NODES, N_CLASSES), jnp.float32),
    )(aggp, dinv, b, g, be, wout, bout)


def kernel(x, edge_index, Ws, bs, gammas, betas, Wout, bout):
    src = edge_index[0].astype(jnp.int32)
    dst = edge_index[1].astype(jnp.int32)
    loop = jnp.arange(N_NODES, dtype=jnp.int32)
    src_full = jnp.concatenate([src, loop])
    dst_full = jnp.concatenate([dst, loop])
    e_all = N_EDGES + N_NODES
    ept = NBLK * B
    lo = e_all // NW
    hi = lo + 1
    n_hi = e_all - NW * lo
    pos = jnp.arange(E_PAD, dtype=jnp.int32)
    w = pos // ept
    k = pos % ept
    r_w = jnp.where(w < n_hi, hi, lo)
    base = jnp.where(w < n_hi, w * hi, n_hi * hi + (w - n_hi) * lo)
    is_real = k < r_w
    take = jnp.where(is_real, base + k, 0)
    src_a = jnp.where(is_real, src_full[take],
                      k % N_NODES).reshape(NW, NBLK, B)
    dst_a = jnp.where(is_real, dst_full[take],
                      N_NODES + (k % (N_PAD - N_NODES))).reshape(NW, NBLK, B)

    ones_r = jnp.ones((B, 16), ROW_F32)
    zer16 = jnp.zeros((STRIPE, 16), ROW_F32)
    zer128 = jnp.zeros((STRIPE, D), ROW_F32)

    degp = _deg_call(dst_a, ones_r, zer16).reshape(2, N_PAD, 16)
    t, dinv = _tc_first(degp, x, Ws[0])

    aggp = None
    for i in range(N_LAYERS):
        aggp = _agg_call(t, src_a, dst_a, zer128).reshape(2, N_PAD, D)
        if i + 1 < N_LAYERS:
            t = _tc_mid(aggp, dinv, bs[i][None, :], gammas[i][None, :],
                        betas[i][None, :], Ws[i + 1])

    i = N_LAYERS - 1
    return _tc_last(aggp, dinv, bs[i][None, :], gammas[i][None, :],
                    betas[i][None, :], Wout, bout[None, :])


# speedup vs baseline: 6.1672x; 1.2815x over previous
import functools

import jax
import jax.numpy as jnp
from jax import lax
from jax.experimental import pallas as pl
from jax.experimental.pallas import tpu as pltpu, tpu_sc as plsc

N_NODES = 10000
N_EDGES = 320000
D = 128
N_CLASSES = 64
N_LAYERS = 5
EPS = 1e-5

NW = 32
B = 88
NBLK = 120
NBUF = 4
G = 8
NG = NBLK // G
ROUNDS = NBLK // NBUF
E_PAD = NW * NBLK * B
N_PAD = 10112
STRIPE = N_PAD // 16
ROW_F32 = jnp.float32

_mesh = plsc.VectorSubcoreMesh(
    core_axis_name="c", subcore_axis_name="s", num_cores=2, num_subcores=16)


def _deg_body(dstp_hbm, ones_hbm, zeros_hbm, out_hbm, dst_v, ones_v, sem, acc_sh):
    c = lax.axis_index("c")
    s = lax.axis_index("s")
    wid = c * 16 + s
    pltpu.sync_copy(dstp_hbm.at[wid], dst_v)
    pltpu.sync_copy(ones_hbm, ones_v)
    pltpu.sync_copy(zeros_hbm, acc_sh.at[pl.ds(s * STRIPE, STRIPE)])
    plsc.subcore_barrier()

    DEG_ASYNC = True
    if DEG_ASYNC:
        K = 8

        def blk(j, carry):
            pltpu.async_copy(ones_v, acc_sh.at[dst_v.at[j]], sem, add=True)

            @pl.when(j >= K)
            def _():
                pltpu.make_async_copy(ones_v, acc_sh.at[dst_v.at[j]],
                                      sem).wait()
            return carry

        lax.fori_loop(0, NBLK, blk, 0)
        for _ in range(K):
            pltpu.make_async_copy(ones_v, acc_sh.at[dst_v.at[0]], sem).wait()
    else:
        def blk(j, carry):
            pltpu.sync_copy(ones_v, acc_sh.at[dst_v.at[j]], add=True)
            return carry

        lax.fori_loop(0, NBLK, blk, 0)
    plsc.subcore_barrier()
    pltpu.sync_copy(
        acc_sh.at[pl.ds(s * STRIPE, STRIPE)],
        out_hbm.at[pl.ds((c * N_PAD + s * STRIPE), STRIPE)],
    )


_deg_call = pl.kernel(
    _deg_body,
    out_type=jax.ShapeDtypeStruct((2 * N_PAD, 16), ROW_F32),
    mesh=_mesh,
    scratch_types=[
        pltpu.VMEM((NBLK, B), jnp.int32),
        pltpu.VMEM((B, 16), ROW_F32),
        pltpu.SemaphoreType.DMA,
        pltpu.VMEM_SHARED((N_PAD, 16), ROW_F32),
    ],
)


def _agg_body(t_hbm, srcp_hbm, dstp_hbm, zeros_hbm, out_hbm,
              sidx_v, didx_v, rows0, rows1, rows2, rows3,
              g0, g1, g2, g3, s0, s1, s2, s3, isem, acc_sh):
    c = lax.axis_index("c")
    s = lax.axis_index("s")
    wid = c * 16 + s
    rows = (rows0, rows1, rows2, rows3)
    gsem = (g0, g1, g2, g3)
    ssem = (s0, s1, s2, s3)

    pltpu.sync_copy(srcp_hbm.at[wid, pl.ds(0, G)], sidx_v.at[pl.ds(0, G)])
    pltpu.sync_copy(dstp_hbm.at[wid, pl.ds(0, G)], didx_v.at[pl.ds(0, G)])
    pltpu.async_copy(srcp_hbm.at[wid, pl.ds(G, G)], sidx_v.at[pl.ds(G, G)],
                     isem)
    pltpu.async_copy(dstp_hbm.at[wid, pl.ds(G, G)], didx_v.at[pl.ds(G, G)],
                     isem)

    pltpu.sync_copy(zeros_hbm, acc_sh.at[pl.ds(s * STRIPE, STRIPE)])
    plsc.subcore_barrier()

    RPG = G // NBUF

    RING = True
    if RING:
        for b in range(NBUF):
            pltpu.async_copy(t_hbm.at[sidx_v.at[b]], rows[b], gsem[b])

    def rnd(i, carry):
        g = i // RPG
        phase = i % RPG

        @pl.when(jnp.logical_and(phase == RPG - 1, g + 1 < NG))
        def _():
            pltpu.make_async_copy(srcp_hbm.at[wid, pl.ds(0, G)],
                                  sidx_v.at[pl.ds(0, G)], isem).wait()
            pltpu.make_async_copy(dstp_hbm.at[wid, pl.ds(0, G)],
                                  didx_v.at[pl.ds(0, G)], isem).wait()

        @pl.when(jnp.logical_and(phase == 0,
                                 jnp.logical_and(g >= 1, g + 1 < NG)))
        def _():
            off = pl.multiple_of(((g + 1) % 2) * G, G)
            src_off = pl.multiple_of((g + 1) * G, G)
            pltpu.async_copy(srcp_hbm.at[wid, pl.ds(src_off, G)],
                             sidx_v.at[pl.ds(off, G)], isem)
            pltpu.async_copy(dstp_hbm.at[wid, pl.ds(src_off, G)],
                             didx_v.at[pl.ds(off, G)], isem)

        if RING:
            for b in range(NBUF):
                j = i * NBUF + b
                jj = j % (2 * G)
                pltpu.make_async_copy(t_hbm.at[sidx_v.at[jj]], rows[b],
                                      gsem[b]).wait()
                pltpu.async_copy(rows[b], acc_sh.at[didx_v.at[jj]], ssem[b],
                                 add=True)

            @pl.when(i + 1 < ROUNDS)
            def _():
                for b in range(NBUF):
                    j = i * NBUF + b
                    jj = j % (2 * G)
                    jn = (j + NBUF) % (2 * G)
                    pltpu.make_async_copy(rows[b], acc_sh.at[didx_v.at[jj]],
                                          ssem[b]).wait()
                    pltpu.async_copy(t_hbm.at[sidx_v.at[jn]], rows[b],
                                     gsem[b])
        else:
            for b in range(NBUF):
                j = i * NBUF + b
                jj = j % (2 * G)
                pltpu.async_copy(t_hbm.at[sidx_v.at[jj]], rows[b], gsem[b])
                pltpu.make_async_copy(t_hbm.at[sidx_v.at[jj]], rows[b],
                                      gsem[b]).wait()
                pltpu.sync_copy(rows[b], acc_sh.at[didx_v.at[jj]], add=True)
        return carry

    lax.fori_loop(0, ROUNDS, rnd, 0)
    if RING:
        for b in range(NBUF):
            jj = ((ROUNDS - 1) * NBUF + b) % (2 * G)
            pltpu.make_async_copy(rows[b], acc_sh.at[didx_v.at[jj]],
                                  ssem[b]).wait()
    plsc.subcore_barrier()
    pltpu.sync_copy(
        acc_sh.at[pl.ds(s * STRIPE, STRIPE)],
        out_hbm.at[pl.ds((c * N_PAD + s * STRIPE), STRIPE)],
    )


_agg_call = pl.kernel(
    _agg_body,
    out_type=jax.ShapeDtypeStruct((2 * N_PAD, D), ROW_F32),
    mesh=_mesh,
    scratch_types=[
        pltpu.VMEM((2 * G, B), jnp.int32),
        pltpu.VMEM((2 * G, B), jnp.int32),
        pltpu.VMEM((B, D), ROW_F32),
        pltpu.VMEM((B, D), ROW_F32),
        pltpu.VMEM((B, D), ROW_F32),
        pltpu.VMEM((B, D), ROW_F32),
        pltpu.SemaphoreType.DMA,
        pltpu.SemaphoreType.DMA,
        pltpu.SemaphoreType.DMA,
        pltpu.SemaphoreType.DMA,
        pltpu.SemaphoreType.DMA,
        pltpu.SemaphoreType.DMA,
        pltpu.SemaphoreType.DMA,
        pltpu.SemaphoreType.DMA,
        pltpu.SemaphoreType.DMA,
        pltpu.VMEM_SHARED((N_PAD, D), ROW_F32),
    ],
)


R = 1000


def _tca_body(degp_ref, x_ref, w_ref, t_ref, dinv_ref):
    dsum = degp_ref[0] + degp_ref[1]
    deg = jnp.sum(dsum, axis=-1, keepdims=True) * (1.0 / 16.0)
    dinv = lax.rsqrt(deg)
    t = jnp.dot(x_ref[...], w_ref[...], preferred_element_type=jnp.float32)
    t_ref[...] = t * dinv
    dinv_ref[...] = dinv


def _tc_first(degp, x, w0):
    return pl.pallas_call(
        _tca_body,
        grid=(N_NODES // R,),
        in_specs=[
            pl.BlockSpec((2, R, 16), lambda i: (0, i, 0)),
            pl.BlockSpec((R, D), lambda i: (i, 0)),
            pl.BlockSpec((D, D), lambda i: (0, 0)),
        ],
        out_specs=[
            pl.BlockSpec((R, D), lambda i: (i, 0)),
            pl.BlockSpec((R, 1), lambda i: (i, 0)),
        ],
        out_shape=[
            jax.ShapeDtypeStruct((N_NODES, D), jnp.float32),
            jax.ShapeDtypeStruct((N_NODES, 1), jnp.float32),
        ],
    )(degp, x, w0)


def _ln_relu(p_ref, dinv_ref, b_ref, g_ref, be_ref):
    h = (p_ref[0] + p_ref[1]) * dinv_ref[...] + b_ref[...]
    mu = jnp.mean(h, axis=-1, keepdims=True)
    hc = h - mu
    var = jnp.mean(hc * hc, axis=-1, keepdims=True)
    h = hc * lax.rsqrt(var + EPS) * g_ref[...] + be_ref[...]
    return jnp.maximum(h, 0.0)


def _tcb_body(p_ref, dinv_ref, b_ref, g_ref, be_ref, w_ref, out_ref):
    h = _ln_relu(p_ref, dinv_ref, b_ref, g_ref, be_ref)
    t = jnp.dot(h, w_ref[...], preferred_element_type=jnp.float32)
    out_ref[...] = t * dinv_ref[...]


def _tc_mid(aggp, dinv, b, g, be, w):
    return pl.pallas_call(
        _tcb_body,
        grid=(N_NODES // R,),
        in_specs=[
            pl.BlockSpec((2, R, D), lambda i: (0, i, 0)),
            pl.BlockSpec((R, 1), lambda i: (i, 0)),
            pl.BlockSpec((1, D), lambda i: (0, 0)),
            pl.BlockSpec((1, D), lambda i: (0, 0)),
            pl.BlockSpec((1, D), lambda i: (0, 0)),
            pl.BlockSpec((D, D), lambda i: (0, 0)),
        ],
        out_specs=pl.BlockSpec((R, D), lambda i: (i, 0)),
        out_shape=jax.ShapeDtypeStruct((N_NODES, D), jnp.float32),
    )(aggp, dinv, b, g, be, w)


def _tcc_body(p_ref, dinv_ref, b_ref, g_ref, be_ref, w_ref, bo_ref, out_ref):
    h = _ln_relu(p_ref, dinv_ref, b_ref, g_ref, be_ref)
    out_ref[...] = (
        jnp.dot(h, w_ref[...], preferred_element_type=jnp.float32) + bo_ref[...]
    )


def _tc_last(aggp, dinv, b, g, be, wout, bout):
    return pl.pallas_call(
        _tcc_body,
        grid=(N_NODES // R,),
        in_specs=[
            pl.BlockSpec((2, R, D), lambda i: (0, i, 0)),
            pl.BlockSpec((R, 1), lambda i: (i, 0)),
            pl.BlockSpec((1, D), lambda i: (0, 0)),
            pl.BlockSpec((1, D), lambda i: (0, 0)),
            pl.BlockSpec((1, D), lambda i: (0, 0)),
            pl.BlockSpec((D, N_CLASSES), lambda i: (0, 0)),
            pl.BlockSpec((1, N_CLASSES), lambda i: (0, 0)),
        ],
        out_specs=pl.BlockSpec((R, N_CLASSES), lambda i: (i, 0)),
        out_shape=jax.ShapeDtypeStruct((N_NODES, N_CLASSES), jnp.float32),
    )(aggp, dinv, b, g, be, wout, bout)


def kernel(x, edge_index, Ws, bs, gammas, betas, Wout, bout):
    src = edge_index[0].astype(jnp.int32)
    dst = edge_index[1].astype(jnp.int32)
    loop = jnp.arange(N_NODES, dtype=jnp.int32)
    src_full = jnp.concatenate([src, loop])
    dst_full = jnp.concatenate([dst, loop])
    e_all = N_EDGES + N_NODES
    ept = NBLK * B
    lo = e_all // NW
    hi = lo + 1
    n_hi = e_all - NW * lo
    pos = jnp.arange(E_PAD, dtype=jnp.int32)
    w = pos // ept
    k = pos % ept
    r_w = jnp.where(w < n_hi, hi, lo)
    base = jnp.where(w < n_hi, w * hi, n_hi * hi + (w - n_hi) * lo)
    is_real = k < r_w
    take = jnp.where(is_real, base + k, 0)
    src_a = jnp.where(is_real, src_full[take],
                      k % N_NODES).reshape(NW, NBLK, B)
    dst_a = jnp.where(is_real, dst_full[take],
                      N_NODES + (k % (N_PAD - N_NODES))).reshape(NW, NBLK, B)

    ones_r = jnp.ones((B, 16), ROW_F32)
    zer16 = jnp.zeros((STRIPE, 16), ROW_F32)
    zer128 = jnp.zeros((STRIPE, D), ROW_F32)

    degp = _deg_call(dst_a, ones_r, zer16).reshape(2, N_PAD, 16)
    t, dinv = _tc_first(degp, x, Ws[0])

    aggp = None
    for i in range(N_LAYERS):
        aggp = _agg_call(t, src_a, dst_a, zer128).reshape(2, N_PAD, D)
        if i + 1 < N_LAYERS:
            t = _tc_mid(aggp, dinv, bs[i][None, :], gammas[i][None, :],
                        betas[i][None, :], Ws[i + 1])

    i = N_LAYERS - 1
    return _tc_last(aggp, dinv, bs[i][None, :], gammas[i][None, :],
                    betas[i][None, :], Wout, bout[None, :])


# speedup vs baseline: 6.9777x; 1.1314x over previous
import functools

import jax
import jax.numpy as jnp
from jax import lax
from jax.experimental import pallas as pl
from jax.experimental.pallas import tpu as pltpu, tpu_sc as plsc

N_NODES = 10000
N_EDGES = 320000
D = 128
N_CLASSES = 64
N_LAYERS = 5
EPS = 1e-5

NW = 32
B = 80
NBLK = 136
NBUF = 4
G = 8
NG = NBLK // G
ROUNDS = NBLK // NBUF
E_PAD = NW * NBLK * B
N_PAD = 10112
STRIPE = N_PAD // 16
ROW_F32 = jnp.float32

_mesh = plsc.VectorSubcoreMesh(
    core_axis_name="c", subcore_axis_name="s", num_cores=2, num_subcores=16)


def _deg_body(dstp_hbm, ones_hbm, zeros_hbm, out_hbm, dst_v, ones_v, sem, acc_sh):
    c = lax.axis_index("c")
    s = lax.axis_index("s")
    wid = c * 16 + s
    pltpu.sync_copy(dstp_hbm.at[wid], dst_v)
    pltpu.sync_copy(ones_hbm, ones_v)
    pltpu.sync_copy(zeros_hbm, acc_sh.at[pl.ds(s * STRIPE, STRIPE)])
    plsc.subcore_barrier()

    DEG_ASYNC = True
    if DEG_ASYNC:
        K = 8

        def blk(j, carry):
            pltpu.async_copy(ones_v, acc_sh.at[dst_v.at[j]], sem, add=True)

            @pl.when(j >= K)
            def _():
                pltpu.make_async_copy(ones_v, acc_sh.at[dst_v.at[j]],
                                      sem).wait()
            return carry

        lax.fori_loop(0, NBLK, blk, 0)
        for _ in range(K):
            pltpu.make_async_copy(ones_v, acc_sh.at[dst_v.at[0]], sem).wait()
    else:
        def blk(j, carry):
            pltpu.sync_copy(ones_v, acc_sh.at[dst_v.at[j]], add=True)
            return carry

        lax.fori_loop(0, NBLK, blk, 0)
    plsc.subcore_barrier()
    pltpu.sync_copy(
        acc_sh.at[pl.ds(s * STRIPE, STRIPE)],
        out_hbm.at[c, pl.ds(s * STRIPE, STRIPE)],
    )


_deg_call = pl.kernel(
    _deg_body,
    out_type=jax.ShapeDtypeStruct((2, N_PAD, 16), ROW_F32),
    mesh=_mesh,
    scratch_types=[
        pltpu.VMEM((NBLK, B), jnp.int32),
        pltpu.VMEM((B, 16), ROW_F32),
        pltpu.SemaphoreType.DMA,
        pltpu.VMEM_SHARED((N_PAD, 16), ROW_F32),
    ],
)


def _agg_body(t_hbm, srcp_hbm, dstp_hbm, zeros_hbm, out_hbm,
              sidx_v, didx_v, rows0, rows1, rows2, rows3,
              g0, g1, g2, g3, s0, s1, s2, s3, isem, acc_sh):
    c = lax.axis_index("c")
    s = lax.axis_index("s")
    wid = c * 16 + s
    rows = (rows0, rows1, rows2, rows3)
    gsem = (g0, g1, g2, g3)
    ssem = (s0, s1, s2, s3)

    pltpu.sync_copy(srcp_hbm.at[wid, pl.ds(0, G)], sidx_v.at[pl.ds(0, G)])
    pltpu.sync_copy(dstp_hbm.at[wid, pl.ds(0, G)], didx_v.at[pl.ds(0, G)])
    pltpu.async_copy(srcp_hbm.at[wid, pl.ds(G, G)], sidx_v.at[pl.ds(G, G)],
                     isem)
    pltpu.async_copy(dstp_hbm.at[wid, pl.ds(G, G)], didx_v.at[pl.ds(G, G)],
                     isem)

    pltpu.sync_copy(zeros_hbm, acc_sh.at[pl.ds(s * STRIPE, STRIPE)])
    plsc.subcore_barrier()

    RPG = G // NBUF

    RING = True
    if RING:
        for b in range(NBUF):
            pltpu.async_copy(t_hbm.at[sidx_v.at[b]], rows[b], gsem[b])

    def rnd(i, carry):
        g = i // RPG
        phase = i % RPG

        @pl.when(jnp.logical_and(phase == RPG - 1, g + 1 < NG))
        def _():
            pltpu.make_async_copy(srcp_hbm.at[wid, pl.ds(0, G)],
                                  sidx_v.at[pl.ds(0, G)], isem).wait()
            pltpu.make_async_copy(dstp_hbm.at[wid, pl.ds(0, G)],
                                  didx_v.at[pl.ds(0, G)], isem).wait()

        @pl.when(jnp.logical_and(phase == 0,
                                 jnp.logical_and(g >= 1, g + 1 < NG)))
        def _():
            off = pl.multiple_of(((g + 1) % 2) * G, G)
            src_off = pl.multiple_of((g + 1) * G, G)
            pltpu.async_copy(srcp_hbm.at[wid, pl.ds(src_off, G)],
                             sidx_v.at[pl.ds(off, G)], isem)
            pltpu.async_copy(dstp_hbm.at[wid, pl.ds(src_off, G)],
                             didx_v.at[pl.ds(off, G)], isem)

        if RING:
            for b in range(NBUF):
                j = i * NBUF + b
                jj = j % (2 * G)
                pltpu.make_async_copy(t_hbm.at[sidx_v.at[jj]], rows[b],
                                      gsem[b]).wait()
                pltpu.async_copy(rows[b], acc_sh.at[didx_v.at[jj]], ssem[b],
                                 add=True)

            @pl.when(i + 1 < ROUNDS)
            def _():
                for b in range(NBUF):
                    j = i * NBUF + b
                    jj = j % (2 * G)
                    jn = (j + NBUF) % (2 * G)
                    pltpu.make_async_copy(rows[b], acc_sh.at[didx_v.at[jj]],
                                          ssem[b]).wait()
                    pltpu.async_copy(t_hbm.at[sidx_v.at[jn]], rows[b],
                                     gsem[b])
        else:
            for b in range(NBUF):
                j = i * NBUF + b
                jj = j % (2 * G)
                pltpu.async_copy(t_hbm.at[sidx_v.at[jj]], rows[b], gsem[b])
                pltpu.make_async_copy(t_hbm.at[sidx_v.at[jj]], rows[b],
                                      gsem[b]).wait()
                pltpu.sync_copy(rows[b], acc_sh.at[didx_v.at[jj]], add=True)
        return carry

    lax.fori_loop(0, ROUNDS, rnd, 0)
    if RING:
        for b in range(NBUF):
            jj = ((ROUNDS - 1) * NBUF + b) % (2 * G)
            pltpu.make_async_copy(rows[b], acc_sh.at[didx_v.at[jj]],
                                  ssem[b]).wait()
    plsc.subcore_barrier()
    pltpu.sync_copy(
        acc_sh.at[pl.ds(s * STRIPE, STRIPE)],
        out_hbm.at[c, pl.ds(s * STRIPE, STRIPE)],
    )


_agg_call = pl.kernel(
    _agg_body,
    out_type=jax.ShapeDtypeStruct((2, N_PAD, D), ROW_F32),
    mesh=_mesh,
    scratch_types=[
        pltpu.VMEM((2 * G, B), jnp.int32),
        pltpu.VMEM((2 * G, B), jnp.int32),
        pltpu.VMEM((B, D), ROW_F32),
        pltpu.VMEM((B, D), ROW_F32),
        pltpu.VMEM((B, D), ROW_F32),
        pltpu.VMEM((B, D), ROW_F32),
        pltpu.SemaphoreType.DMA,
        pltpu.SemaphoreType.DMA,
        pltpu.SemaphoreType.DMA,
        pltpu.SemaphoreType.DMA,
        pltpu.SemaphoreType.DMA,
        pltpu.SemaphoreType.DMA,
        pltpu.SemaphoreType.DMA,
        pltpu.SemaphoreType.DMA,
        pltpu.SemaphoreType.DMA,
        pltpu.VMEM_SHARED((N_PAD, D), ROW_F32),
    ],
)


R = 1000


def _tca_body(degp_ref, x_ref, w_ref, t_ref, dinv_ref):
    dsum = degp_ref[0] + degp_ref[1]
    deg = jnp.sum(dsum, axis=-1, keepdims=True) * (1.0 / 16.0)
    dinv = lax.rsqrt(deg)
    t = jnp.dot(x_ref[...], w_ref[...], preferred_element_type=jnp.float32)
    t_ref[...] = t * dinv
    dinv_ref[...] = dinv


def _tc_first(degp, x, w0):
    return pl.pallas_call(
        _tca_body,
        grid=(N_NODES // R,),
        in_specs=[
            pl.BlockSpec((2, R, 16), lambda i: (0, i, 0)),
            pl.BlockSpec((R, D), lambda i: (i, 0)),
            pl.BlockSpec((D, D), lambda i: (0, 0)),
        ],
        out_specs=[
            pl.BlockSpec((R, D), lambda i: (i, 0)),
            pl.BlockSpec((R, 1), lambda i: (i, 0)),
        ],
        out_shape=[
            jax.ShapeDtypeStruct((N_NODES, D), jnp.float32),
            jax.ShapeDtypeStruct((N_NODES, 1), jnp.float32),
        ],
    )(degp, x, w0)


def _ln_relu(p_ref, dinv_ref, b_ref, g_ref, be_ref):
    h = (p_ref[0] + p_ref[1]) * dinv_ref[...] + b_ref[...]
    mu = jnp.mean(h, axis=-1, keepdims=True)
    hc = h - mu
    var = jnp.mean(hc * hc, axis=-1, keepdims=True)
    h = hc * lax.rsqrt(var + EPS) * g_ref[...] + be_ref[...]
    return jnp.maximum(h, 0.0)


def _tcb_body(p_ref, dinv_ref, b_ref, g_ref, be_ref, w_ref, out_ref):
    h = _ln_relu(p_ref, dinv_ref, b_ref, g_ref, be_ref)
    t = jnp.dot(h, w_ref[...], preferred_element_type=jnp.float32)
    out_ref[...] = t * dinv_ref[...]


def _tc_mid(aggp, dinv, b, g, be, w):
    return pl.pallas_call(
        _tcb_body,
        grid=(N_NODES // R,),
        in_specs=[
            pl.BlockSpec((2, R, D), lambda i: (0, i, 0)),
            pl.BlockSpec((R, 1), lambda i: (i, 0)),
            pl.BlockSpec((1, D), lambda i: (0, 0)),
            pl.BlockSpec((1, D), lambda i: (0, 0)),
            pl.BlockSpec((1, D), lambda i: (0, 0)),
            pl.BlockSpec((D, D), lambda i: (0, 0)),
        ],
        out_specs=pl.BlockSpec((R, D), lambda i: (i, 0)),
        out_shape=jax.ShapeDtypeStruct((N_NODES, D), jnp.float32),
    )(aggp, dinv, b, g, be, w)


def _tcc_body(p_ref, dinv_ref, b_ref, g_ref, be_ref, w_ref, bo_ref, out_ref):
    h = _ln_relu(p_ref, dinv_ref, b_ref, g_ref, be_ref)
    out_ref[...] = (
        jnp.dot(h, w_ref[...], preferred_element_type=jnp.float32) + bo_ref[...]
    )


def _tc_last(aggp, dinv, b, g, be, wout, bout):
    return pl.pallas_call(
        _tcc_body,
        grid=(N_NODES // R,),
        in_specs=[
            pl.BlockSpec((2, R, D), lambda i: (0, i, 0)),
            pl.BlockSpec((R, 1), lambda i: (i, 0)),
            pl.BlockSpec((1, D), lambda i: (0, 0)),
            pl.BlockSpec((1, D), lambda i: (0, 0)),
            pl.BlockSpec((1, D), lambda i: (0, 0)),
            pl.BlockSpec((D, N_CLASSES), lambda i: (0, 0)),
            pl.BlockSpec((1, N_CLASSES), lambda i: (0, 0)),
        ],
        out_specs=pl.BlockSpec((R, N_CLASSES), lambda i: (i, 0)),
        out_shape=jax.ShapeDtypeStruct((N_NODES, N_CLASSES), jnp.float32),
    )(aggp, dinv, b, g, be, wout, bout)


def kernel(x, edge_index, Ws, bs, gammas, betas, Wout, bout):
    src = edge_index[0].astype(jnp.int32)
    dst = edge_index[1].astype(jnp.int32)
    ept = NBLK * B
    n_loop_pad = NW * 320 - N_NODES
    n_tail = ept - N_EDGES // NW - 320
    loop = jnp.arange(N_NODES, dtype=jnp.int32)
    dum_src = (jnp.arange(NW * n_tail + n_loop_pad, dtype=jnp.int32)
               % N_NODES)
    dum_dst = N_NODES + (jnp.arange(NW * n_tail + n_loop_pad,
                                    dtype=jnp.int32) % (N_PAD - N_NODES))
    loop_s = jnp.concatenate([loop, dum_src[NW * n_tail:]]).reshape(NW, 320)
    loop_d = jnp.concatenate([loop, dum_dst[NW * n_tail:]]).reshape(NW, 320)
    src_a = jnp.concatenate(
        [src.reshape(NW, N_EDGES // NW), loop_s,
         dum_src[:NW * n_tail].reshape(NW, n_tail)],
        axis=1).reshape(NW, NBLK, B)
    dst_a = jnp.concatenate(
        [dst.reshape(NW, N_EDGES // NW), loop_d,
         dum_dst[:NW * n_tail].reshape(NW, n_tail)],
        axis=1).reshape(NW, NBLK, B)

    ones_r = jnp.ones((B, 16), ROW_F32)
    zer16 = jnp.zeros((STRIPE, 16), ROW_F32)
    zer128 = jnp.zeros((STRIPE, D), ROW_F32)

    degp = _deg_call(dst_a, ones_r, zer16)
    t, dinv = _tc_first(degp, x, Ws[0])

    aggp = None
    for i in range(N_LAYERS):
        aggp = _agg_call(t, src_a, dst_a, zer128)
        if i + 1 < N_LAYERS:
            t = _tc_mid(aggp, dinv, bs[i][None, :], gammas[i][None, :],
                        betas[i][None, :], Ws[i + 1])

    i = N_LAYERS - 1
    return _tc_last(aggp, dinv, bs[i][None, :], gammas[i][None, :],
                    betas[i][None, :], Wout, bout[None, :])


# speedup vs baseline: 6.9908x; 1.0019x over previous
import functools

import jax
import jax.numpy as jnp
from jax import lax
from jax.experimental import pallas as pl
from jax.experimental.pallas import tpu as pltpu, tpu_sc as plsc

N_NODES = 10000
N_EDGES = 320000
D = 128
N_CLASSES = 64
N_LAYERS = 5
EPS = 1e-5

NW = 32
B = 80
NBLK = 136
NBUF = 4
G = 8
NG = NBLK // G
ROUNDS = NBLK // NBUF
E_PAD = NW * NBLK * B
N_PAD = 10112
STRIPE = N_PAD // 16
ROW_F32 = jnp.float32

_mesh = plsc.VectorSubcoreMesh(
    core_axis_name="c", subcore_axis_name="s", num_cores=2, num_subcores=16)


def _deg_body(dstp_hbm, ones_hbm, zeros_hbm, out_hbm, dst_v, ones_v, sem, acc_sh):
    c = lax.axis_index("c")
    s = lax.axis_index("s")
    wid = c * 16 + s
    pltpu.sync_copy(dstp_hbm.at[wid], dst_v)
    pltpu.sync_copy(ones_hbm, ones_v)
    pltpu.sync_copy(zeros_hbm, acc_sh.at[pl.ds(s * STRIPE, STRIPE)])
    plsc.subcore_barrier()

    K = 8

    def blk(j, carry):
        pltpu.async_copy(ones_v, acc_sh.at[dst_v.at[j]], sem, add=True)

        @pl.when(j >= K)
        def _():
            pltpu.make_async_copy(ones_v, acc_sh.at[dst_v.at[j]],
                                  sem).wait()
        return carry

    lax.fori_loop(0, NBLK, blk, 0)
    for _ in range(K):
        pltpu.make_async_copy(ones_v, acc_sh.at[dst_v.at[0]], sem).wait()
    plsc.subcore_barrier()
    pltpu.sync_copy(
        acc_sh.at[pl.ds(s * STRIPE, STRIPE)],
        out_hbm.at[c, pl.ds(s * STRIPE, STRIPE)],
    )


_deg_call = pl.kernel(
    _deg_body,
    out_type=jax.ShapeDtypeStruct((2, N_PAD, 16), ROW_F32),
    mesh=_mesh,
    scratch_types=[
        pltpu.VMEM((NBLK, B), jnp.int32),
        pltpu.VMEM((B, 16), ROW_F32),
        pltpu.SemaphoreType.DMA,
        pltpu.VMEM_SHARED((N_PAD, 16), ROW_F32),
    ],
)


def _agg_body(t_hbm, srcp_hbm, dstp_hbm, zeros_hbm, out_hbm,
              sidx_v, didx_v, rows0, rows1, rows2, rows3,
              g0, g1, g2, g3, s0, s1, s2, s3, isem, acc_sh):
    c = lax.axis_index("c")
    s = lax.axis_index("s")
    wid = c * 16 + s
    rows = (rows0, rows1, rows2, rows3)
    gsem = (g0, g1, g2, g3)
    ssem = (s0, s1, s2, s3)

    pltpu.sync_copy(srcp_hbm.at[wid, pl.ds(0, G)], sidx_v.at[pl.ds(0, G)])
    pltpu.sync_copy(dstp_hbm.at[wid, pl.ds(0, G)], didx_v.at[pl.ds(0, G)])
    pltpu.async_copy(srcp_hbm.at[wid, pl.ds(G, G)], sidx_v.at[pl.ds(G, G)],
                     isem)
    pltpu.async_copy(dstp_hbm.at[wid, pl.ds(G, G)], didx_v.at[pl.ds(G, G)],
                     isem)

    pltpu.sync_copy(zeros_hbm, acc_sh.at[pl.ds(s * STRIPE, STRIPE)])
    plsc.subcore_barrier()

    RPG = G // NBUF

    for b in range(NBUF):
        pltpu.async_copy(t_hbm.at[sidx_v.at[b]], rows[b], gsem[b])

    def rnd(i, carry):
        g = i // RPG
        phase = i % RPG

        @pl.when(jnp.logical_and(phase == RPG - 1, g + 1 < NG))
        def _():
            pltpu.make_async_copy(srcp_hbm.at[wid, pl.ds(0, G)],
                                  sidx_v.at[pl.ds(0, G)], isem).wait()
            pltpu.make_async_copy(dstp_hbm.at[wid, pl.ds(0, G)],
                                  didx_v.at[pl.ds(0, G)], isem).wait()

        @pl.when(jnp.logical_and(phase == 0,
                                 jnp.logical_and(g >= 1, g + 1 < NG)))
        def _():
            off = pl.multiple_of(((g + 1) % 2) * G, G)
            src_off = pl.multiple_of((g + 1) * G, G)
            pltpu.async_copy(srcp_hbm.at[wid, pl.ds(src_off, G)],
                             sidx_v.at[pl.ds(off, G)], isem)
            pltpu.async_copy(dstp_hbm.at[wid, pl.ds(src_off, G)],
                             didx_v.at[pl.ds(off, G)], isem)

        for b in range(NBUF):
            j = i * NBUF + b
            jj = j % (2 * G)
            pltpu.make_async_copy(t_hbm.at[sidx_v.at[jj]], rows[b],
                                  gsem[b]).wait()
            pltpu.async_copy(rows[b], acc_sh.at[didx_v.at[jj]], ssem[b],
                             add=True)

        @pl.when(i + 1 < ROUNDS)
        def _():
            for b in range(NBUF):
                j = i * NBUF + b
                jj = j % (2 * G)
                jn = (j + NBUF) % (2 * G)
                pltpu.make_async_copy(rows[b], acc_sh.at[didx_v.at[jj]],
                                      ssem[b]).wait()
                pltpu.async_copy(t_hbm.at[sidx_v.at[jn]], rows[b],
                                 gsem[b])
        return carry

    lax.fori_loop(0, ROUNDS, rnd, 0)
    for b in range(NBUF):
        jj = ((ROUNDS - 1) * NBUF + b) % (2 * G)
        pltpu.make_async_copy(rows[b], acc_sh.at[didx_v.at[jj]],
                              ssem[b]).wait()
    plsc.subcore_barrier()
    pltpu.sync_copy(
        acc_sh.at[pl.ds(s * STRIPE, STRIPE)],
        out_hbm.at[c, pl.ds(s * STRIPE, STRIPE)],
    )


_agg_call = pl.kernel(
    _agg_body,
    out_type=jax.ShapeDtypeStruct((2, N_PAD, D), ROW_F32),
    mesh=_mesh,
    scratch_types=[
        pltpu.VMEM((2 * G, B), jnp.int32),
        pltpu.VMEM((2 * G, B), jnp.int32),
        pltpu.VMEM((B, D), ROW_F32),
        pltpu.VMEM((B, D), ROW_F32),
        pltpu.VMEM((B, D), ROW_F32),
        pltpu.VMEM((B, D), ROW_F32),
        pltpu.SemaphoreType.DMA,
        pltpu.SemaphoreType.DMA,
        pltpu.SemaphoreType.DMA,
        pltpu.SemaphoreType.DMA,
        pltpu.SemaphoreType.DMA,
        pltpu.SemaphoreType.DMA,
        pltpu.SemaphoreType.DMA,
        pltpu.SemaphoreType.DMA,
        pltpu.SemaphoreType.DMA,
        pltpu.VMEM_SHARED((N_PAD, D), ROW_F32),
    ],
)


R = 1000


def _tca_body(degp_ref, x_ref, w_ref, t_ref, dinv_ref):
    dsum = degp_ref[0] + degp_ref[1]
    deg = jnp.sum(dsum, axis=-1, keepdims=True) * (1.0 / 16.0)
    dinv = lax.rsqrt(deg)
    t = jnp.dot(x_ref[...], w_ref[...], preferred_element_type=jnp.float32)
    t_ref[...] = t * dinv
    dinv_ref[...] = dinv


def _tc_first(degp, x, w0):
    return pl.pallas_call(
        _tca_body,
        grid=(N_NODES // R,),
        in_specs=[
            pl.BlockSpec((2, R, 16), lambda i: (0, i, 0)),
            pl.BlockSpec((R, D), lambda i: (i, 0)),
            pl.BlockSpec((D, D), lambda i: (0, 0)),
        ],
        out_specs=[
            pl.BlockSpec((R, D), lambda i: (i, 0)),
            pl.BlockSpec((R, 1), lambda i: (i, 0)),
        ],
        out_shape=[
            jax.ShapeDtypeStruct((N_NODES, D), jnp.float32),
            jax.ShapeDtypeStruct((N_NODES, 1), jnp.float32),
        ],
    )(degp, x, w0)


def _ln_relu(p_ref, dinv_ref, b_ref, g_ref, be_ref):
    h = (p_ref[0] + p_ref[1]) * dinv_ref[...] + b_ref[...]
    mu = jnp.mean(h, axis=-1, keepdims=True)
    hc = h - mu
    var = jnp.mean(hc * hc, axis=-1, keepdims=True)
    h = hc * lax.rsqrt(var + EPS) * g_ref[...] + be_ref[...]
    return jnp.maximum(h, 0.0)


def _tcb_body(p_ref, dinv_ref, b_ref, g_ref, be_ref, w_ref, out_ref):
    h = _ln_relu(p_ref, dinv_ref, b_ref, g_ref, be_ref)
    t = jnp.dot(h, w_ref[...], preferred_element_type=jnp.float32)
    out_ref[...] = t * dinv_ref[...]


def _tc_mid(aggp, dinv, b, g, be, w):
    return pl.pallas_call(
        _tcb_body,
        grid=(N_NODES // R,),
        in_specs=[
            pl.BlockSpec((2, R, D), lambda i: (0, i, 0)),
            pl.BlockSpec((R, 1), lambda i: (i, 0)),
            pl.BlockSpec((1, D), lambda i: (0, 0)),
            pl.BlockSpec((1, D), lambda i: (0, 0)),
            pl.BlockSpec((1, D), lambda i: (0, 0)),
            pl.BlockSpec((D, D), lambda i: (0, 0)),
        ],
        out_specs=pl.BlockSpec((R, D), lambda i: (i, 0)),
        out_shape=jax.ShapeDtypeStruct((N_NODES, D), jnp.float32),
    )(aggp, dinv, b, g, be, w)


def _tcc_body(p_ref, dinv_ref, b_ref, g_ref, be_ref, w_ref, bo_ref, out_ref):
    h = _ln_relu(p_ref, dinv_ref, b_ref, g_ref, be_ref)
    out_ref[...] = (
        jnp.dot(h, w_ref[...], preferred_element_type=jnp.float32) + bo_ref[...]
    )


def _tc_last(aggp, dinv, b, g, be, wout, bout):
    return pl.pallas_call(
        _tcc_body,
        grid=(N_NODES // R,),
        in_specs=[
            pl.BlockSpec((2, R, D), lambda i: (0, i, 0)),
            pl.BlockSpec((R, 1), lambda i: (i, 0)),
            pl.BlockSpec((1, D), lambda i: (0, 0)),
            pl.BlockSpec((1, D), lambda i: (0, 0)),
            pl.BlockSpec((1, D), lambda i: (0, 0)),
            pl.BlockSpec((D, N_CLASSES), lambda i: (0, 0)),
            pl.BlockSpec((1, N_CLASSES), lambda i: (0, 0)),
        ],
        out_specs=pl.BlockSpec((R, N_CLASSES), lambda i: (i, 0)),
        out_shape=jax.ShapeDtypeStruct((N_NODES, N_CLASSES), jnp.float32),
    )(aggp, dinv, b, g, be, wout, bout)


def kernel(x, edge_index, Ws, bs, gammas, betas, Wout, bout):
    src = edge_index[0].astype(jnp.int32)
    dst = edge_index[1].astype(jnp.int32)
    ept = NBLK * B
    n_loop_pad = NW * 320 - N_NODES
    n_tail = ept - N_EDGES // NW - 320
    loop = jnp.arange(N_NODES, dtype=jnp.int32)
    dum_src = (jnp.arange(NW * n_tail + n_loop_pad, dtype=jnp.int32)
               % N_NODES)
    dum_dst = N_NODES + (jnp.arange(NW * n_tail + n_loop_pad,
                                    dtype=jnp.int32) % (N_PAD - N_NODES))
    loop_s = jnp.concatenate([loop, dum_src[NW * n_tail:]]).reshape(NW, 320)
    loop_d = jnp.concatenate([loop, dum_dst[NW * n_tail:]]).reshape(NW, 320)
    src_a = jnp.concatenate(
        [src.reshape(NW, N_EDGES // NW), loop_s,
         dum_src[:NW * n_tail].reshape(NW, n_tail)],
        axis=1).reshape(NW, NBLK, B)
    dst_a = jnp.concatenate(
        [dst.reshape(NW, N_EDGES // NW), loop_d,
         dum_dst[:NW * n_tail].reshape(NW, n_tail)],
        axis=1).reshape(NW, NBLK, B)

    ones_r = jnp.ones((B, 16), ROW_F32)
    zer16 = jnp.zeros((STRIPE, 16), ROW_F32)
    zer128 = jnp.zeros((STRIPE, D), ROW_F32)

    degp = _deg_call(dst_a, ones_r, zer16)
    t, dinv = _tc_first(degp, x, Ws[0])

    aggp = None
    for i in range(N_LAYERS):
        aggp = _agg_call(t, src_a, dst_a, zer128)
        if i + 1 < N_LAYERS:
            t = _tc_mid(aggp, dinv, bs[i][None, :], gammas[i][None, :],
                        betas[i][None, :], Ws[i + 1])

    i = N_LAYERS - 1
    return _tc_last(aggp, dinv, bs[i][None, :], gammas[i][None, :],
                    betas[i][None, :], Wout, bout[None, :])
